# Initial kernel scaffold; baseline (speedup 1.0000x reference)
import jax
import jax.numpy as jnp
from jax.experimental import pallas as pl


def kernel(x, WQ, WK, WV, WZ_w, WZ_b, expand_w, expand_b, restore_w, restore_b):
    raise NotImplementedError("write your pallas kernel here")


# speedup vs baseline: 6.9773x; 6.9773x over previous
import jax
import jax.numpy as jnp
import numpy as np
from jax.experimental import pallas as pl
from jax.experimental.pallas import tpu as pltpu

_S = 4
_HEAD = 8
_QK = 64
_BR = 8
_BF = jnp.bfloat16
_F32 = jnp.float32


def _dot(a, b):
    return jnp.dot(a, b, preferred_element_type=_F32)


def _body(x_ref, rzj_ref, e4_ref, selp_ref,
          wqt_ref, wkt_ref, wvt_ref, wzt_ref, wzb_ref,
          ewt_ref, eb_ref, rwt_ref, rb_ref, o_ref):
    x = x_ref[...]
    C, D = x.shape[1], x.shape[2]
    nc = C // _S
    PS = _S * C
    P2 = _BR * PS

    xb = x.astype(_BF)
    x2 = xb.reshape(P2, D)

    kp = _dot(x2, wkt_ref[...]).astype(_BF)
    vp = _dot(x2, wvt_ref[...]).astype(_BF)
    hq = kp.shape[1]

    xr = x.reshape(_BR, _S, C, D).sum(axis=1).astype(_BF)
    xm = jnp.concatenate(
        [_dot(rzj_ref[...], xr[t]) for t in range(_BR)], axis=0)
    q = _dot(xm.astype(_BF), wqt_ref[...])

    qe = jnp.concatenate(
        [_dot(e4_ref[...], q[t * nc:(t + 1) * nc].astype(_BF))
         for t in range(_BR)], axis=0).astype(_BF)

    prod = qe.reshape(_BR, 1, C, hq) * kp.reshape(_BR, _S, C, hq)
    prod = prod.reshape(P2, hq)
    psum = (prod[:, 0:128] + prod[:, 128:256]
            + prod[:, 256:384] + prod[:, 384:512])
    scores = _dot(psum, selp_ref[...])
    e = jnp.exp(scores)

    er = e.reshape(_BR, _S, C, 128).sum(axis=1).astype(_BF)
    esum = jnp.concatenate(
        [_dot(rzj_ref[...], er[t]) for t in range(_BR)], axis=0)
    rcp = 1.0 / esum

    e512 = pltpu.repeat(e.astype(_BF), 4, axis=1)
    wv = (e512 * vp).reshape(_BR, _S, C, hq).sum(axis=1)
    zw = jnp.concatenate(
        [_dot(rzj_ref[...], wv[t].astype(_BF)) for t in range(_BR)],
        axis=0)
    zw = zw * pltpu.repeat(rcp, 4, axis=1)

    z = (_dot(zw.astype(_BF), wzt_ref[...]) + wzb_ref[...]).astype(_BF)
    h = jnp.maximum(_dot(z, ewt_ref[...]) + eb_ref[...], 0.0).astype(_BF)
    out = jnp.maximum(_dot(h, rwt_ref[...]) + rb_ref[...], 0.0)
    o_ref[...] = out.reshape(_BR, nc, -1)


def kernel(x, WQ, WK, WV, WZ_w, WZ_b, expand_w, expand_b, restore_w, restore_b):
    b, r, c, d = x.shape
    nr, nc = r // _S, c // _S
    hq = WQ.shape[0]
    hid = expand_w.shape[0]
    dout = WZ_w.shape[0]

    wi = np.arange(nc)
    ci = np.arange(c)
    rzj = (ci[None, :] // _S == wi[:, None]).astype(np.float32)
    e4 = (ci[:, None] // _S == wi[None, :]).astype(np.float32)
    li = np.arange(128)
    selp = ((li[:, None] % _HEAD == li[None, :] % _HEAD)
            .astype(np.float32) / np.sqrt(_QK).astype(np.float32))

    iperm = (np.arange(hq) % _HEAD) * _QK + np.arange(hq) // _HEAD

    xf = x.reshape(b * r, c, d)
    grid = (b * nr // _BR,)
    cw = lambda shape: pl.BlockSpec(shape, lambda i: tuple(0 for _ in shape))

    out = pl.pallas_call(
        _body,
        grid=grid,
        in_specs=[
            pl.BlockSpec((_BR * _S, c, d), lambda i: (i, 0, 0)),
            cw((nc, c)),
            cw((c, nc)),
            cw((128, 128)),
            cw((d, hq)),
            cw((d, hq)),
            cw((d, hq)),
            cw((hq, dout)),
            cw((1, dout)),
            cw((dout, hid)),
            cw((1, hid)),
            cw((hid, dout)),
            cw((1, dout)),
        ],
        out_specs=pl.BlockSpec((_BR, nc, dout), lambda i: (i, 0, 0)),
        out_shape=jax.ShapeDtypeStruct((b * nr, nc, dout), jnp.float32),
        compiler_params=pltpu.CompilerParams(
            dimension_semantics=("parallel",),
            vmem_limit_bytes=56 * 1024 * 1024,
        ),
    )(
        xf,
        jnp.asarray(rzj, _BF), jnp.asarray(e4, _BF), jnp.asarray(selp, _BF),
        (WQ.T / (_S * _S)).astype(_BF)[:, iperm],
        WK.T.astype(_BF)[:, iperm],
        WV.T.astype(_BF)[:, iperm],
        WZ_w.T.astype(_BF)[iperm, :],
        WZ_b.reshape(1, dout),
        expand_w.T.astype(_BF),
        expand_b.reshape(1, hid),
        restore_w.T.astype(_BF),
        restore_b.reshape(1, dout),
    )
    return out.reshape(b, nr, nc, dout)
